# Initial kernel scaffold; baseline (speedup 1.0000x reference)
import functools

import jax
import jax.numpy as jnp
from jax import lax
from jax.experimental import pallas as pl
from jax.experimental.pallas import tpu as pltpu
from jax.experimental.pallas import tpu_sc as plsc

_B, _L, _D = 16384, 50, 64
_NC, _NS = 2, 16
_NW = _NC * _NS
_SPW = _B // _NW
_C = 8
_ROWS = _C * _L
_NCHUNK = _SPW // _C


def _sc_body(idx_hbm, wts_hbm, table_hbm, out_hbm, idx_v, wts_v, rows_v,
             out_v, sem):
    cid = lax.axis_index("c")
    sid = lax.axis_index("s")
    wid = sid * _NC + cid
    base = wid * _SPW

    def chunk_body(j, _):
        b0 = base + j * _C
        pltpu.sync_copy(idx_hbm.at[pl.ds(b0 * _L, _ROWS)], idx_v)
        pltpu.sync_copy(wts_hbm.at[pl.ds(b0 * _L, _ROWS)], wts_v)
        pltpu.async_copy(table_hbm.at[idx_v], rows_v, sem).wait()

        def sample_body(s, _):
            def l_body(l, carry):
                a0, a1, a2, a3, ws = carry
                p = s * _L + l
                wb = plsc.load_gather(wts_v, [jnp.full((16,), p, jnp.int32)])
                a0 = a0 + rows_v[p, pl.ds(0, 16)] * wb
                a1 = a1 + rows_v[p, pl.ds(16, 16)] * wb
                a2 = a2 + rows_v[p, pl.ds(32, 16)] * wb
                a3 = a3 + rows_v[p, pl.ds(48, 16)] * wb
                return (a0, a1, a2, a3, ws + wb)

            z = jnp.zeros((16,), jnp.float32)
            a0, a1, a2, a3, ws = lax.fori_loop(0, _L, l_body, (z, z, z, z, z))
            inv = 1.0 / ws
            out_v[s, pl.ds(0, 16)] = a0 * inv
            out_v[s, pl.ds(16, 16)] = a1 * inv
            out_v[s, pl.ds(32, 16)] = a2 * inv
            out_v[s, pl.ds(48, 16)] = a3 * inv
            return 0

        lax.fori_loop(0, _C, sample_body, 0)
        pltpu.sync_copy(out_v, out_hbm.at[pl.ds(b0, _C)])
        return 0

    lax.fori_loop(0, _NCHUNK, chunk_body, 0)


@jax.jit
def _run(idx_flat, wts_flat, table):
    mesh = plsc.VectorSubcoreMesh(core_axis_name="c", subcore_axis_name="s")
    k = pl.kernel(
        _sc_body,
        mesh=mesh,
        out_type=jax.ShapeDtypeStruct((_B, _D), jnp.float32),
        scratch_types=[
            pltpu.VMEM((_ROWS,), jnp.int32),
            pltpu.VMEM((_ROWS,), jnp.float32),
            pltpu.VMEM((_ROWS, _D), jnp.float32),
            pltpu.VMEM((_C, _D), jnp.float32),
            pltpu.SemaphoreType.DMA,
        ],
    )
    return k(idx_flat, wts_flat, table)


def kernel(lookup_tensor, weights_tensor, weight):
    idx_flat = lookup_tensor.reshape(-1)
    wts_flat = weights_tensor.reshape(-1)
    return _run(idx_flat, wts_flat, weight)


# speedup vs baseline: 2.3112x; 2.3112x over previous
import functools

import jax
import jax.numpy as jnp
from jax import lax
from jax.experimental import pallas as pl
from jax.experimental.pallas import tpu as pltpu
from jax.experimental.pallas import tpu_sc as plsc

_B, _L, _D = 16384, 50, 64
_NC, _NS = 2, 16
_NW = _NC * _NS
_SPW = _B // _NW
_C = 8
_ROWS = _C * _L
_NCHUNK = _SPW // _C


def _sc_body(idx_hbm, wts_hbm, table_hbm, out_hbm, idx_v, wts_v, rows_v,
             out_v, sem):
    cid = lax.axis_index("c")
    sid = lax.axis_index("s")
    wid = sid * _NC + cid
    base = wid * _SPW

    def chunk_body(j, _):
        b0 = base + j * _C
        pltpu.sync_copy(idx_hbm.at[pl.ds(b0 * _L, _ROWS)], idx_v)
        pltpu.sync_copy(wts_hbm.at[pl.ds(b0 * _L, _ROWS)],
                        wts_v.at[pl.ds(0, _ROWS)])
        pltpu.async_copy(table_hbm.at[idx_v], rows_v, sem).wait()

        def sample_body(s, _):
            p0 = s * _L
            wv = [wts_v[pl.ds(p0 + 16 * c, 16)] for c in range(4)]
            acc = [jnp.zeros((16,), jnp.float32) for _ in range(4)]
            ws = jnp.zeros((16,), jnp.float32)
            for l in range(_L):
                wb = jnp.full((16,), wv[l // 16][l % 16], jnp.float32)
                ws = ws + wb
                p = p0 + l
                acc[0] = acc[0] + rows_v[p, pl.ds(0, 16)] * wb
                acc[1] = acc[1] + rows_v[p, pl.ds(16, 16)] * wb
                acc[2] = acc[2] + rows_v[p, pl.ds(32, 16)] * wb
                acc[3] = acc[3] + rows_v[p, pl.ds(48, 16)] * wb
            inv = 1.0 / ws
            out_v[s, pl.ds(0, 16)] = acc[0] * inv
            out_v[s, pl.ds(16, 16)] = acc[1] * inv
            out_v[s, pl.ds(32, 16)] = acc[2] * inv
            out_v[s, pl.ds(48, 16)] = acc[3] * inv
            return 0

        lax.fori_loop(0, _C, sample_body, 0)
        pltpu.sync_copy(out_v, out_hbm.at[pl.ds(b0, _C)])
        return 0

    lax.fori_loop(0, _NCHUNK, chunk_body, 0)


@jax.jit
def _run(idx_flat, wts_flat, table):
    mesh = plsc.VectorSubcoreMesh(core_axis_name="c", subcore_axis_name="s")
    k = pl.kernel(
        _sc_body,
        mesh=mesh,
        compiler_params=pltpu.CompilerParams(use_tc_tiling_on_sc=False),
        out_type=jax.ShapeDtypeStruct((_B, _D), jnp.float32),
        scratch_types=[
            pltpu.VMEM((_ROWS,), jnp.int32),
            pltpu.VMEM((_ROWS + 16,), jnp.float32),
            pltpu.VMEM((_ROWS, _D), jnp.float32),
            pltpu.VMEM((_C, _D), jnp.float32),
            pltpu.SemaphoreType.DMA,
        ],
    )
    return k(idx_flat, wts_flat, table)


def kernel(lookup_tensor, weights_tensor, weight):
    idx_flat = lookup_tensor.reshape(-1)
    wts_flat = weights_tensor.reshape(-1)
    return _run(idx_flat, wts_flat, weight)


# speedup vs baseline: 2.6052x; 1.1272x over previous
import functools

import jax
import jax.numpy as jnp
from jax import lax
from jax.experimental import pallas as pl
from jax.experimental.pallas import tpu as pltpu
from jax.experimental.pallas import tpu_sc as plsc

_B, _L, _D = 16384, 50, 64
_NC, _NS = 2, 16
_NW = _NC * _NS
_SPW = _B // _NW
_C = 8
_ROWS = _C * _L
_NCHUNK = _SPW // _C


def _sc_body(idx_hbm, wts_hbm, table_hbm, out_hbm, idx_v0, idx_v1, wts_v0,
             wts_v1, rows_v0, rows_v1, out_v0, out_v1, sem0, sem1):
    cid = lax.axis_index("c")
    sid = lax.axis_index("s")
    wid = sid * _NC + cid
    base = wid * _SPW

    def start(j, idx_v, wts_v, rows_v, sem):
        b0 = base + j * _C
        pltpu.sync_copy(idx_hbm.at[pl.ds(b0 * _L, _ROWS)], idx_v)
        pltpu.sync_copy(wts_hbm.at[pl.ds(b0 * _L, _ROWS)],
                        wts_v.at[pl.ds(0, _ROWS)])
        pltpu.async_copy(table_hbm.at[idx_v], rows_v, sem)

    def wait(idx_v, rows_v, sem):
        pltpu.make_async_copy(table_hbm.at[idx_v], rows_v, sem).wait()

    def compute(j, wts_v, rows_v, out_v):
        b0 = base + j * _C

        def sample_body(s, _):
            p0 = s * _L
            wv = [wts_v[pl.ds(p0 + 16 * c, 16)] for c in range(4)]
            acc = [jnp.zeros((16,), jnp.float32) for _ in range(4)]
            ws = jnp.zeros((16,), jnp.float32)
            for l in range(_L):
                wb = jnp.full((16,), wv[l // 16][l % 16], jnp.float32)
                ws = ws + wb
                p = p0 + l
                acc[0] = acc[0] + rows_v[p, pl.ds(0, 16)] * wb
                acc[1] = acc[1] + rows_v[p, pl.ds(16, 16)] * wb
                acc[2] = acc[2] + rows_v[p, pl.ds(32, 16)] * wb
                acc[3] = acc[3] + rows_v[p, pl.ds(48, 16)] * wb
            inv = 1.0 / ws
            out_v[s, pl.ds(0, 16)] = acc[0] * inv
            out_v[s, pl.ds(16, 16)] = acc[1] * inv
            out_v[s, pl.ds(32, 16)] = acc[2] * inv
            out_v[s, pl.ds(48, 16)] = acc[3] * inv
            return 0

        lax.fori_loop(0, _C, sample_body, 0)
        pltpu.sync_copy(out_v, out_hbm.at[pl.ds(b0, _C)])

    start(0, idx_v0, wts_v0, rows_v0, sem0)

    def pair_body(p, _):
        j0 = 2 * p
        start(j0 + 1, idx_v1, wts_v1, rows_v1, sem1)
        wait(idx_v0, rows_v0, sem0)
        compute(j0, wts_v0, rows_v0, out_v0)
        jn = jnp.minimum(j0 + 2, _NCHUNK - 1)
        start(jn, idx_v0, wts_v0, rows_v0, sem0)
        wait(idx_v1, rows_v1, sem1)
        compute(j0 + 1, wts_v1, rows_v1, out_v1)
        return 0

    lax.fori_loop(0, _NCHUNK // 2, pair_body, 0)
    wait(idx_v0, rows_v0, sem0)


@jax.jit
def _run(idx_flat, wts_flat, table):
    mesh = plsc.VectorSubcoreMesh(core_axis_name="c", subcore_axis_name="s")
    k = pl.kernel(
        _sc_body,
        mesh=mesh,
        compiler_params=pltpu.CompilerParams(use_tc_tiling_on_sc=False),
        out_type=jax.ShapeDtypeStruct((_B, _D), jnp.float32),
        scratch_types=[
            pltpu.VMEM((_ROWS,), jnp.int32),
            pltpu.VMEM((_ROWS,), jnp.int32),
            pltpu.VMEM((_ROWS + 16,), jnp.float32),
            pltpu.VMEM((_ROWS + 16,), jnp.float32),
            pltpu.VMEM((_ROWS, _D), jnp.float32),
            pltpu.VMEM((_ROWS, _D), jnp.float32),
            pltpu.VMEM((_C, _D), jnp.float32),
            pltpu.VMEM((_C, _D), jnp.float32),
            pltpu.SemaphoreType.DMA,
            pltpu.SemaphoreType.DMA,
        ],
    )
    return k(idx_flat, wts_flat, table)


def kernel(lookup_tensor, weights_tensor, weight):
    idx_flat = lookup_tensor.reshape(-1)
    wts_flat = weights_tensor.reshape(-1)
    return _run(idx_flat, wts_flat, weight)
